# baseline (speedup 1.0000x reference)
import functools

import jax
import jax.numpy as jnp
import numpy as np
from jax import lax
from jax.experimental import pallas as pl
from jax.experimental.pallas import tpu as pltpu
from jax.experimental.pallas import tpu_sc as plsc

_PHI = (1.0 + 5.0 ** 0.5) / 2.0
_SQRT_PHI = float(np.sqrt(_PHI))
_D = 768
_E = 6
_KINDS = ('arith', 'general', 'geom', 'quantum', 'general', 'general')
_N = 8192
_NPAIR = 2 * _N
_TBLK = 256
_NTILE = _NPAIR // _TBLK
_NWORK = _NTILE + _E
_NW = 32
_CHUNK = 64
_NCH = _N // _NW // _CHUNK
_OW = _D + 128
_NEG = -1e30


def _layernorm(v, g, b, eps=1e-5):
    m = jnp.mean(v, axis=-1, keepdims=True)
    c = v - m
    var = jnp.mean(c * c, axis=-1, keepdims=True)
    return c * jax.lax.rsqrt(var + eps) * g + b


def _gelu_exact(v):
    return 0.5 * v * (1.0 + jax.lax.erf(v * float(1.0 / np.sqrt(2.0))))


def _act1(kind, h):
    if kind == 'arith':
        return jax.nn.relu(h)
    if kind == 'quantum':
        return jnp.tanh(h)
    return _gelu_exact(h)


def _act2(kind, o):
    if kind == 'quantum':
        return jnp.tanh(o)
    if kind == 'geom':
        return _gelu_exact(o)
    return o


def _dot_t(a, b):
    return jax.lax.dot_general(a, b, (((1,), (1,)), ((), ())),
                               preferred_element_type=jnp.float32)


def _dot(a, b):
    return jax.lax.dot_general(a, b, (((1,), (0,)), ((), ())),
                               preferred_element_type=jnp.float32)


def _gate_kernel(x_ref, wg_ref, bg_ref, e0_ref, e1_ref, g0_ref, g1_ref):
    x = x_ref[...]
    logits = _dot_t(x, wg_ref[...]) + bg_ref[...]
    col = jax.lax.broadcasted_iota(jnp.int32, logits.shape, 1)
    valid = col < _E
    lm = jnp.where(valid, logits, _NEG)
    m0 = jnp.max(lm, axis=1, keepdims=True)
    is0 = jnp.logical_and(lm == m0, valid)
    arg0 = jnp.min(jnp.where(is0, col, 127), axis=1, keepdims=True)
    lm1 = jnp.where(col == arg0, _NEG, lm)
    m1 = jnp.max(lm1, axis=1, keepdims=True)
    is1 = jnp.logical_and(lm1 == m1, jnp.logical_and(valid, col != arg0))
    arg1 = jnp.min(jnp.where(is1, col, 127), axis=1, keepdims=True)
    g0 = 1.0 / (1.0 + jnp.exp((m1 - m0) * _SQRT_PHI))
    e0_ref[...] = arg0
    e1_ref[...] = arg1
    g0_ref[...] = g0
    g1_ref[...] = 1.0 - g0


def _gate(x2d, wg, bg):
    grid = (_N // _TBLK,)
    spec_c1 = pl.BlockSpec((_TBLK, 1), lambda i: (i, 0))
    return pl.pallas_call(
        _gate_kernel,
        grid=grid,
        in_specs=[pl.BlockSpec((_TBLK, _D), lambda i: (i, 0)),
                  pl.BlockSpec(wg.shape, lambda i: (0, 0)),
                  pl.BlockSpec(bg.shape, lambda i: (0, 0))],
        out_specs=[spec_c1] * 4,
        out_shape=[jax.ShapeDtypeStruct((_N, 1), jnp.int32),
                   jax.ShapeDtypeStruct((_N, 1), jnp.int32),
                   jax.ShapeDtypeStruct((_N, 1), jnp.float32),
                   jax.ShapeDtypeStruct((_N, 1), jnp.float32)],
    )(x2d, wg, bg)


def _plan_kernel(e0_ref, e1_ref, pos0_ref, pos1_ref, work_ref):
    e0 = e0_ref[...]
    e1 = e1_ref[...]
    rr, cc = e0.shape
    ui = jax.lax.broadcasted_iota(jnp.int32, (cc, cc), 0)
    uj = jax.lax.broadcasted_iota(jnp.int32, (cc, cc), 1)
    upper = (ui < uj).astype(jnp.float32)
    vi = jax.lax.broadcasted_iota(jnp.int32, (rr, rr), 0)
    vj = jax.lax.broadcasted_iota(jnp.int32, (rr, rr), 1)
    lower = (vj < vi).astype(jnp.float32)

    base = jnp.zeros((1, 1), jnp.float32)
    pos0 = jnp.zeros((rr, cc), jnp.float32)
    pos1 = jnp.zeros((rr, cc), jnp.float32)
    starts, ends = [], []
    for e in range(_E):
        hit0 = (e0 == e)
        hit1 = (e1 == e)
        cnt = hit0.astype(jnp.float32) + hit1.astype(jnp.float32)
        excl = _dot(cnt, upper)
        rowsum = jnp.sum(cnt, axis=1, keepdims=True)
        carry = _dot(lower, rowsum)
        gpos = base + excl + carry
        pos0 = pos0 + hit0.astype(jnp.float32) * gpos
        pos1 = pos1 + hit1.astype(jnp.float32) * gpos
        total = jnp.sum(rowsum, axis=0, keepdims=True)
        starts.append(base)
        base = base + total
        ends.append(base)
    pos0_ref[...] = pos0.astype(jnp.int32)
    pos1_ref[...] = pos1.astype(jnp.int32)

    wcol = jax.lax.broadcasted_iota(jnp.int32, (1, 128), 1)
    ew = jnp.full((1, 128), _E + 1, jnp.int32)
    mw = jnp.full((1, 128), _NTILE - 1, jnp.int32)
    sw = jnp.zeros((1, 128), jnp.int32)
    qw = jnp.zeros((1, 128), jnp.int32)
    running = jnp.zeros((1, 1), jnp.int32)
    for e in range(_E):
        s_i = starts[e].astype(jnp.int32)
        q_i = ends[e].astype(jnp.int32)
        first_t = lax.div(s_i, _TBLK)
        last_t = lax.div(q_i - 1, _TBLK)
        count = jnp.where(q_i > s_i, last_t - first_t + 1, 0)
        sel = jnp.logical_and(wcol >= running, wcol < running + count)
        ew = jnp.where(sel, e, ew)
        mw = jnp.where(sel, first_t + (wcol - running), mw)
        sw = jnp.where(sel, s_i, sw)
        qw = jnp.where(sel, q_i, qw)
        running = running + count
    work_ref[0:1, :] = ew
    work_ref[1:2, :] = mw
    work_ref[2:3, :] = sw
    work_ref[3:4, :] = qw


def _plan(e0m, e1m):
    return pl.pallas_call(
        _plan_kernel,
        out_shape=[jax.ShapeDtypeStruct(e0m.shape, jnp.int32),
                   jax.ShapeDtypeStruct(e0m.shape, jnp.int32),
                   jax.ShapeDtypeStruct((4, 128), jnp.int32)],
    )(e0m, e1m)


def _dispatch_sc(x32, p0w, p1w):
    w32 = _D // 2
    mesh = plsc.VectorSubcoreMesh(core_axis_name="c", subcore_axis_name="s")

    @functools.partial(
        pl.kernel,
        out_type=jax.ShapeDtypeStruct((_NPAIR, w32), jnp.int32),
        mesh=mesh,
        scratch_types=[
            pltpu.VMEM((_NCH, _CHUNK), jnp.int32),
            pltpu.VMEM((_NCH, _CHUNK), jnp.int32),
            pltpu.VMEM((_CHUNK, w32), jnp.int32),
            pltpu.SemaphoreType.DMA,
        ],
    )
    def k(x_hbm, p0_hbm, p1_hbm, out_hbm, idx0_v, idx1_v, rows_v, sem):
        wid = lax.axis_index("s") * 2 + lax.axis_index("c")
        base = wid * (_N // _NW)
        pltpu.sync_copy(p0_hbm.at[wid], idx0_v)
        pltpu.sync_copy(p1_hbm.at[wid], idx1_v)
        for j in range(_NCH):
            pltpu.sync_copy(x_hbm.at[pl.ds(base + j * _CHUNK, _CHUNK)], rows_v)
            pltpu.async_copy(rows_v, out_hbm.at[idx0_v.at[j]], sem).wait()
            pltpu.async_copy(rows_v, out_hbm.at[idx1_v.at[j]], sem).wait()

    return k(x32, p0w, p1w)


def _experts_kernel(eparams, ew, mw, sw, qw, xs_ref, out_ref):
    i = pl.program_id(0)
    e_id = ew[i]
    m = mw[i]
    prev = mw[jnp.maximum(i - 1, 0)]
    first = jnp.logical_or(i == 0, m != prev)

    @pl.when(first)
    def _():
        out_ref[...] = jnp.zeros_like(out_ref)

    rows = m * _TBLK + jax.lax.broadcasted_iota(jnp.int32, (_TBLK, 1), 0)
    mask = jnp.logical_and(rows >= sw[i], rows < qw[i]).astype(jnp.float32)
    x = xs_ref[...].astype(jnp.float32)
    for e in range(_E):
        @pl.when(e_id == e)
        def _(e=e):
            w1, b1, w2, b2, wc1, bc1, wc2, bc2, spec, lng, lnb = eparams[e]
            z = x + spec[...]
            h = _act1(_KINDS[e], _dot_t(z, w1[...]) + b1[...])
            o = _act2(_KINDS[e], _dot_t(h, w2[...]) + b2[...])
            o = _layernorm(o, lng[...], lnb[...])
            r = jax.nn.relu(_dot_t(o, wc1[...]) + bc1[...])
            clin = jnp.sum(r * wc2[...], axis=1, keepdims=True) + bc2[...]
            ce = jax.nn.sigmoid(clin)
            out_ref[:, :_D] += mask * o
            out_ref[:, _D:] += mask * jnp.broadcast_to(ce, (_TBLK, _OW - _D))


def _experts(xs, flat, ew, mw, sw, qw):
    nflat = len(flat)

    def body(ew_r, mw_r, sw_r, qw_r, xs_ref, *refs):
        eparams = [refs[11 * e:11 * e + 11] for e in range(_E)]
        _experts_kernel(eparams, ew_r, mw_r, sw_r, qw_r, xs_ref, refs[nflat])

    grid_spec = pltpu.PrefetchScalarGridSpec(
        num_scalar_prefetch=4,
        grid=(_NWORK,),
        in_specs=[pl.BlockSpec((_TBLK, _D), lambda i, ew, mw, sw, qw: (mw[i], 0))]
        + [pl.BlockSpec(w.shape, lambda i, *_: (0,) * w.ndim) for w in flat],
        out_specs=pl.BlockSpec((_TBLK, _OW), lambda i, ew, mw, sw, qw: (mw[i], 0)),
    )
    return pl.pallas_call(
        body,
        grid_spec=grid_spec,
        out_shape=jax.ShapeDtypeStruct((_NPAIR, _OW), jnp.float32),
    )(ew, mw, sw, qw, xs, *flat)


def _combine_gather_sc(outs, p0w, p1w):
    mesh = plsc.VectorSubcoreMesh(core_axis_name="c", subcore_axis_name="s")

    @functools.partial(
        pl.kernel,
        out_type=(jax.ShapeDtypeStruct((_N, _OW), jnp.float32),
                  jax.ShapeDtypeStruct((_N, _OW), jnp.float32)),
        mesh=mesh,
        scratch_types=[
            pltpu.VMEM((_NCH, _CHUNK), jnp.int32),
            pltpu.VMEM((_NCH, _CHUNK), jnp.int32),
            pltpu.VMEM((_CHUNK, _OW), jnp.float32),
            pltpu.SemaphoreType.DMA,
        ],
    )
    def k(outs_hbm, p0_hbm, p1_hbm, g0_hbm, g1_hbm, idx0_v, idx1_v, rows_v, sem):
        wid = lax.axis_index("s") * 2 + lax.axis_index("c")
        base = wid * (_N // _NW)
        pltpu.sync_copy(p0_hbm.at[wid], idx0_v)
        pltpu.sync_copy(p1_hbm.at[wid], idx1_v)
        for j in range(_NCH):
            dst = pl.ds(base + j * _CHUNK, _CHUNK)
            pltpu.async_copy(outs_hbm.at[idx0_v.at[j]], rows_v, sem).wait()
            pltpu.sync_copy(rows_v, g0_hbm.at[dst])
            pltpu.async_copy(outs_hbm.at[idx1_v.at[j]], rows_v, sem).wait()
            pltpu.sync_copy(rows_v, g1_hbm.at[dst])

    return k(outs, p0w, p1w)


def _combine_kernel(r0_ref, r1_ref, g0_ref, g1_ref, wcm_ref, bcm_ref,
                    cg_ref, cb_ref, out_ref, conf_ref):
    g0 = g0_ref[...]
    g1 = g1_ref[...]
    r0 = r0_ref[...]
    r1 = r1_ref[...]
    combined = g0 * r0[:, :_D] + g1 * r1[:, :_D]
    conf = g0 * r0[:, _D:_D + 1] + g1 * r1[:, _D:_D + 1]
    y = _dot_t(combined, wcm_ref[...]) + bcm_ref[...]
    y = _layernorm(y, cg_ref[...], cb_ref[...])
    out_ref[...] = y
    conf_ref[...] = jnp.broadcast_to(conf, conf_ref.shape)


def _combine(g0rows, g1rows, gw0, gw1, wcm, bcm, cg, cb):
    grid = (_N // _TBLK,)
    spec_row = pl.BlockSpec((_TBLK, _OW), lambda i: (i, 0))
    spec_c1 = pl.BlockSpec((_TBLK, 1), lambda i: (i, 0))
    return pl.pallas_call(
        _combine_kernel,
        grid=grid,
        in_specs=[spec_row, spec_row, spec_c1, spec_c1,
                  pl.BlockSpec(wcm.shape, lambda i: (0, 0)),
                  pl.BlockSpec(bcm.shape, lambda i: (0, 0)),
                  pl.BlockSpec(cg.shape, lambda i: (0, 0)),
                  pl.BlockSpec(cb.shape, lambda i: (0, 0))],
        out_specs=[pl.BlockSpec((_TBLK, _D), lambda i: (i, 0)),
                   pl.BlockSpec((_TBLK, 128), lambda i: (i, 0))],
        out_shape=[jax.ShapeDtypeStruct((_N, _D), jnp.float32),
                   jax.ShapeDtypeStruct((_N, 128), jnp.float32)],
    )(g0rows, g1rows, gw0, gw1, wcm, bcm, cg, cb)


def kernel(x, params):
    b, s, d = x.shape
    x2d = x.reshape(b * s, d)
    gate = params['gate']
    wg = jnp.zeros((128, d), jnp.float32).at[:_E].set(gate['W'])
    bg = jnp.zeros((1, 128), jnp.float32).at[0, :_E].set(gate['b'])

    e0, e1, gw0, gw1 = _gate(x2d, wg, bg)
    pos0, pos1, work = _plan(e0.reshape(_N // 128, 128),
                             e1.reshape(_N // 128, 128))
    ew, mw, sw, qw = (work[j, :_NWORK] for j in range(4))
    p0w = pos0.reshape(_NW, _NCH, _CHUNK)
    p1w = pos1.reshape(_NW, _NCH, _CHUNK)

    x16 = x2d.astype(jnp.bfloat16)
    x32 = jax.lax.bitcast_convert_type(x16.reshape(_N, _D // 2, 2), jnp.int32)
    xs32 = _dispatch_sc(x32, p0w, p1w)
    xs = jax.lax.bitcast_convert_type(xs32, jnp.bfloat16).reshape(_NPAIR, _D)

    flat = []
    for e in range(_E):
        p = params['experts'][e]
        flat += [p['W1'], p['b1'][None, :], p['W2'], p['b2'][None, :],
                 p['Wc1'], p['bc1'][None, :], p['Wc2'], p['bc2'][None, :],
                 p['spec'][None, :], p['ln_g'][None, :], p['ln_b'][None, :]]
    outs = _experts(xs, tuple(flat), ew, mw, sw, qw)

    g0rows, g1rows = _combine_gather_sc(outs, p0w, p1w)

    cmb = params['combiner']
    out2d, conf = _combine(g0rows, g1rows, gw0, gw1, cmb['W'],
                           cmb['b'][None, :], cmb['ln_g'][None, :],
                           cmb['ln_b'][None, :])
    return out2d.reshape(b, s, d), conf[:, 0].reshape(b, s)


# speedup vs baseline: 1.7829x; 1.7829x over previous
import functools

import jax
import jax.numpy as jnp
import numpy as np
from jax import lax
from jax.experimental import pallas as pl
from jax.experimental.pallas import tpu as pltpu
from jax.experimental.pallas import tpu_sc as plsc

_PHI = (1.0 + 5.0 ** 0.5) / 2.0
_SQRT_PHI = float(np.sqrt(_PHI))
_D = 768
_E = 6
_KINDS = ('arith', 'general', 'geom', 'quantum', 'general', 'general')
_N = 8192
_NPAIR = 2 * _N
_TBLK = 256
_NTILE = _NPAIR // _TBLK
_NWORK = _NTILE + _E
_NW = 32
_CHUNK = 64
_NCH = _N // _NW // _CHUNK
_OW = _D + 128
_NEG = -1e30


def _layernorm(v, g, b, eps=1e-5):
    m = jnp.mean(v, axis=-1, keepdims=True)
    c = v - m
    var = jnp.mean(c * c, axis=-1, keepdims=True)
    return c * jax.lax.rsqrt(var + eps) * g + b


def _gelu_exact(v):
    return 0.5 * v * (1.0 + jax.lax.erf(v * float(1.0 / np.sqrt(2.0))))


def _act1(kind, h):
    if kind == 'arith':
        return jax.nn.relu(h)
    if kind == 'quantum':
        return jnp.tanh(h)
    return _gelu_exact(h)


def _act2(kind, o):
    if kind == 'quantum':
        return jnp.tanh(o)
    if kind == 'geom':
        return _gelu_exact(o)
    return o


def _dot_t(a, b):
    return jax.lax.dot_general(a, b, (((1,), (1,)), ((), ())),
                               preferred_element_type=jnp.float32)


def _dot(a, b):
    return jax.lax.dot_general(a, b, (((1,), (0,)), ((), ())),
                               preferred_element_type=jnp.float32)


def _gate_kernel(x_ref, wg_ref, bg_ref, e0_ref, e1_ref, g0_ref, g1_ref):
    x = x_ref[...]
    logits = _dot_t(x, wg_ref[...]) + bg_ref[...]
    col = jax.lax.broadcasted_iota(jnp.int32, logits.shape, 1)
    valid = col < _E
    lm = jnp.where(valid, logits, _NEG)
    m0 = jnp.max(lm, axis=1, keepdims=True)
    is0 = jnp.logical_and(lm == m0, valid)
    arg0 = jnp.min(jnp.where(is0, col, 127), axis=1, keepdims=True)
    lm1 = jnp.where(col == arg0, _NEG, lm)
    m1 = jnp.max(lm1, axis=1, keepdims=True)
    is1 = jnp.logical_and(lm1 == m1, jnp.logical_and(valid, col != arg0))
    arg1 = jnp.min(jnp.where(is1, col, 127), axis=1, keepdims=True)
    g0 = 1.0 / (1.0 + jnp.exp((m1 - m0) * _SQRT_PHI))
    e0_ref[...] = arg0
    e1_ref[...] = arg1
    g0_ref[...] = g0
    g1_ref[...] = 1.0 - g0


def _gate(x2d, wg, bg):
    grid = (_N // _TBLK,)
    spec_c1 = pl.BlockSpec((_TBLK, 1), lambda i: (i, 0))
    return pl.pallas_call(
        _gate_kernel,
        grid=grid,
        in_specs=[pl.BlockSpec((_TBLK, _D), lambda i: (i, 0)),
                  pl.BlockSpec(wg.shape, lambda i: (0, 0)),
                  pl.BlockSpec(bg.shape, lambda i: (0, 0))],
        out_specs=[spec_c1] * 4,
        out_shape=[jax.ShapeDtypeStruct((_N, 1), jnp.int32),
                   jax.ShapeDtypeStruct((_N, 1), jnp.int32),
                   jax.ShapeDtypeStruct((_N, 1), jnp.float32),
                   jax.ShapeDtypeStruct((_N, 1), jnp.float32)],
    )(x2d, wg, bg)


def _plan_kernel(e0_ref, e1_ref, pos0_ref, pos1_ref, work_ref):
    e0 = e0_ref[...]
    e1 = e1_ref[...]
    rr, cc = e0.shape
    ui = jax.lax.broadcasted_iota(jnp.int32, (cc, cc), 0)
    uj = jax.lax.broadcasted_iota(jnp.int32, (cc, cc), 1)
    upper = (ui < uj).astype(jnp.float32)
    vi = jax.lax.broadcasted_iota(jnp.int32, (rr, rr), 0)
    vj = jax.lax.broadcasted_iota(jnp.int32, (rr, rr), 1)
    lower = (vj < vi).astype(jnp.float32)

    base = jnp.zeros((1, 1), jnp.float32)
    pos0 = jnp.zeros((rr, cc), jnp.float32)
    pos1 = jnp.zeros((rr, cc), jnp.float32)
    starts, ends = [], []
    for e in range(_E):
        hit0 = (e0 == e)
        hit1 = (e1 == e)
        cnt = hit0.astype(jnp.float32) + hit1.astype(jnp.float32)
        excl = _dot(cnt, upper)
        rowsum = jnp.sum(cnt, axis=1, keepdims=True)
        carry = _dot(lower, rowsum)
        gpos = base + excl + carry
        pos0 = pos0 + hit0.astype(jnp.float32) * gpos
        pos1 = pos1 + hit1.astype(jnp.float32) * gpos
        total = jnp.sum(rowsum, axis=0, keepdims=True)
        starts.append(base)
        base = base + total
        ends.append(base)
    pos0_ref[...] = pos0.astype(jnp.int32)
    pos1_ref[...] = pos1.astype(jnp.int32)

    wcol = jax.lax.broadcasted_iota(jnp.int32, (1, 128), 1)
    ew = jnp.full((1, 128), _E + 1, jnp.int32)
    mw = jnp.full((1, 128), _NTILE - 1, jnp.int32)
    sw = jnp.zeros((1, 128), jnp.int32)
    qw = jnp.zeros((1, 128), jnp.int32)
    running = jnp.zeros((1, 1), jnp.int32)
    for e in range(_E):
        s_i = starts[e].astype(jnp.int32)
        q_i = ends[e].astype(jnp.int32)
        first_t = lax.div(s_i, _TBLK)
        last_t = lax.div(q_i - 1, _TBLK)
        count = jnp.where(q_i > s_i, last_t - first_t + 1, 0)
        sel = jnp.logical_and(wcol >= running, wcol < running + count)
        ew = jnp.where(sel, e, ew)
        mw = jnp.where(sel, first_t + (wcol - running), mw)
        sw = jnp.where(sel, s_i, sw)
        qw = jnp.where(sel, q_i, qw)
        running = running + count
    work_ref[0:1, :] = ew
    work_ref[1:2, :] = mw
    work_ref[2:3, :] = sw
    work_ref[3:4, :] = qw


def _plan(e0m, e1m):
    return pl.pallas_call(
        _plan_kernel,
        out_shape=[jax.ShapeDtypeStruct(e0m.shape, jnp.int32),
                   jax.ShapeDtypeStruct(e0m.shape, jnp.int32),
                   jax.ShapeDtypeStruct((4, 128), jnp.int32)],
    )(e0m, e1m)


def _dispatch_sc(x2d, p0w, p1w):
    mesh = plsc.VectorSubcoreMesh(core_axis_name="c", subcore_axis_name="s")

    @functools.partial(
        pl.kernel,
        out_type=jax.ShapeDtypeStruct((_NPAIR, _D), jnp.float32),
        mesh=mesh,
        scratch_types=[
            pltpu.VMEM((_NCH, _CHUNK), jnp.int32),
            pltpu.VMEM((_NCH, _CHUNK), jnp.int32),
            pltpu.VMEM((_CHUNK, _D), jnp.float32),
            pltpu.SemaphoreType.DMA,
        ],
    )
    def k(x_hbm, p0_hbm, p1_hbm, out_hbm, idx0_v, idx1_v, rows_v, sem):
        wid = lax.axis_index("s") * 2 + lax.axis_index("c")
        base = wid * (_N // _NW)
        pltpu.sync_copy(p0_hbm.at[wid], idx0_v)
        pltpu.sync_copy(p1_hbm.at[wid], idx1_v)
        for j in range(_NCH):
            pltpu.sync_copy(x_hbm.at[pl.ds(base + j * _CHUNK, _CHUNK)], rows_v)
            pltpu.async_copy(rows_v, out_hbm.at[idx0_v.at[j]], sem).wait()
            pltpu.async_copy(rows_v, out_hbm.at[idx1_v.at[j]], sem).wait()

    return k(x2d, p0w, p1w)


def _experts_kernel(eparams, ew, mw, sw, qw, xs_ref, out_ref):
    i = pl.program_id(0)
    e_id = ew[i]
    m = mw[i]
    prev = mw[jnp.maximum(i - 1, 0)]
    first = jnp.logical_or(i == 0, m != prev)

    @pl.when(first)
    def _():
        out_ref[...] = jnp.zeros_like(out_ref)

    rows = m * _TBLK + jax.lax.broadcasted_iota(jnp.int32, (_TBLK, 1), 0)
    mask = jnp.logical_and(rows >= sw[i], rows < qw[i]).astype(jnp.float32)
    x = xs_ref[...]
    for e in range(_E):
        @pl.when(e_id == e)
        def _(e=e):
            w1, b1, w2, b2, wc1, bc1, wc2, bc2, spec, lng, lnb = eparams[e]
            z = x + spec[...]
            h = _act1(_KINDS[e], _dot_t(z, w1[...]) + b1[...])
            o = _act2(_KINDS[e], _dot_t(h, w2[...]) + b2[...])
            o = _layernorm(o, lng[...], lnb[...])
            r = jax.nn.relu(_dot_t(o, wc1[...]) + bc1[...])
            clin = jnp.sum(r * wc2[...], axis=1, keepdims=True) + bc2[...]
            ce = jax.nn.sigmoid(clin)
            out_ref[:, :_D] += mask * o
            out_ref[:, _D:] += mask * jnp.broadcast_to(ce, (_TBLK, _OW - _D))


def _experts(xs, flat, ew, mw, sw, qw):
    nflat = len(flat)

    def body(ew_r, mw_r, sw_r, qw_r, xs_ref, *refs):
        eparams = [refs[11 * e:11 * e + 11] for e in range(_E)]
        _experts_kernel(eparams, ew_r, mw_r, sw_r, qw_r, xs_ref, refs[nflat])

    grid_spec = pltpu.PrefetchScalarGridSpec(
        num_scalar_prefetch=4,
        grid=(_NWORK,),
        in_specs=[pl.BlockSpec((_TBLK, _D), lambda i, ew, mw, sw, qw: (mw[i], 0))]
        + [pl.BlockSpec(w.shape, lambda i, *_: (0,) * w.ndim) for w in flat],
        out_specs=pl.BlockSpec((_TBLK, _OW), lambda i, ew, mw, sw, qw: (mw[i], 0)),
    )
    return pl.pallas_call(
        body,
        grid_spec=grid_spec,
        out_shape=jax.ShapeDtypeStruct((_NPAIR, _OW), jnp.float32),
    )(ew, mw, sw, qw, xs, *flat)


def _combine_gather_sc(outs, p0w, p1w):
    mesh = plsc.VectorSubcoreMesh(core_axis_name="c", subcore_axis_name="s")

    @functools.partial(
        pl.kernel,
        out_type=(jax.ShapeDtypeStruct((_N, _OW), jnp.float32),
                  jax.ShapeDtypeStruct((_N, _OW), jnp.float32)),
        mesh=mesh,
        scratch_types=[
            pltpu.VMEM((_NCH, _CHUNK), jnp.int32),
            pltpu.VMEM((_NCH, _CHUNK), jnp.int32),
            pltpu.VMEM((_CHUNK, _OW), jnp.float32),
            pltpu.SemaphoreType.DMA,
        ],
    )
    def k(outs_hbm, p0_hbm, p1_hbm, g0_hbm, g1_hbm, idx0_v, idx1_v, rows_v, sem):
        wid = lax.axis_index("s") * 2 + lax.axis_index("c")
        base = wid * (_N // _NW)
        pltpu.sync_copy(p0_hbm.at[wid], idx0_v)
        pltpu.sync_copy(p1_hbm.at[wid], idx1_v)
        for j in range(_NCH):
            dst = pl.ds(base + j * _CHUNK, _CHUNK)
            pltpu.async_copy(outs_hbm.at[idx0_v.at[j]], rows_v, sem).wait()
            pltpu.sync_copy(rows_v, g0_hbm.at[dst])
            pltpu.async_copy(outs_hbm.at[idx1_v.at[j]], rows_v, sem).wait()
            pltpu.sync_copy(rows_v, g1_hbm.at[dst])

    return k(outs, p0w, p1w)


def _combine_kernel(r0_ref, r1_ref, g0_ref, g1_ref, wcm_ref, bcm_ref,
                    cg_ref, cb_ref, out_ref, conf_ref):
    g0 = g0_ref[...]
    g1 = g1_ref[...]
    r0 = r0_ref[...]
    r1 = r1_ref[...]
    combined = g0 * r0[:, :_D] + g1 * r1[:, :_D]
    conf = g0 * r0[:, _D:_D + 1] + g1 * r1[:, _D:_D + 1]
    y = _dot_t(combined, wcm_ref[...]) + bcm_ref[...]
    y = _layernorm(y, cg_ref[...], cb_ref[...])
    out_ref[...] = y
    conf_ref[...] = jnp.broadcast_to(conf, conf_ref.shape)


def _combine(g0rows, g1rows, gw0, gw1, wcm, bcm, cg, cb):
    grid = (_N // _TBLK,)
    spec_row = pl.BlockSpec((_TBLK, _OW), lambda i: (i, 0))
    spec_c1 = pl.BlockSpec((_TBLK, 1), lambda i: (i, 0))
    return pl.pallas_call(
        _combine_kernel,
        grid=grid,
        in_specs=[spec_row, spec_row, spec_c1, spec_c1,
                  pl.BlockSpec(wcm.shape, lambda i: (0, 0)),
                  pl.BlockSpec(bcm.shape, lambda i: (0, 0)),
                  pl.BlockSpec(cg.shape, lambda i: (0, 0)),
                  pl.BlockSpec(cb.shape, lambda i: (0, 0))],
        out_specs=[pl.BlockSpec((_TBLK, _D), lambda i: (i, 0)),
                   pl.BlockSpec((_TBLK, 128), lambda i: (i, 0))],
        out_shape=[jax.ShapeDtypeStruct((_N, _D), jnp.float32),
                   jax.ShapeDtypeStruct((_N, 128), jnp.float32)],
    )(g0rows, g1rows, gw0, gw1, wcm, bcm, cg, cb)


def kernel(x, params):
    b, s, d = x.shape
    x2d = x.reshape(b * s, d)
    gate = params['gate']
    wg = jnp.zeros((128, d), jnp.float32).at[:_E].set(gate['W'])
    bg = jnp.zeros((1, 128), jnp.float32).at[0, :_E].set(gate['b'])

    e0, e1, gw0, gw1 = _gate(x2d, wg, bg)
    pos0, pos1, work = _plan(e0.reshape(_N // 128, 128),
                             e1.reshape(_N // 128, 128))
    ew, mw, sw, qw = (work[j, :_NWORK] for j in range(4))
    p0w = pos0.reshape(_NW, _NCH, _CHUNK)
    p1w = pos1.reshape(_NW, _NCH, _CHUNK)

    xs = _dispatch_sc(x2d, p0w, p1w)

    flat = []
    for e in range(_E):
        p = params['experts'][e]
        flat += [p['W1'], p['b1'][None, :], p['W2'], p['b2'][None, :],
                 p['Wc1'], p['bc1'][None, :], p['Wc2'], p['bc2'][None, :],
                 p['spec'][None, :], p['ln_g'][None, :], p['ln_b'][None, :]]
    outs = _experts(xs, tuple(flat), ew, mw, sw, qw)

    g0rows, g1rows = _combine_gather_sc(outs, p0w, p1w)

    cmb = params['combiner']
    out2d, conf = _combine(g0rows, g1rows, gw0, gw1, cmb['W'],
                           cmb['b'][None, :], cmb['ln_g'][None, :],
                           cmb['ln_b'][None, :])
    return out2d.reshape(b, s, d), conf[:, 0].reshape(b, s)


# speedup vs baseline: 1.8617x; 1.0442x over previous
import functools

import jax
import jax.numpy as jnp
import numpy as np
from jax import lax
from jax.experimental import pallas as pl
from jax.experimental.pallas import tpu as pltpu
from jax.experimental.pallas import tpu_sc as plsc

_PHI = (1.0 + 5.0 ** 0.5) / 2.0
_SQRT_PHI = float(np.sqrt(_PHI))
_D = 768
_E = 6
_KINDS = ('arith', 'general', 'geom', 'quantum', 'general', 'general')
_N = 8192
_NPAIR = 2 * _N
_TBLK = 256
_EBLK = 512
_NTILE = _NPAIR // _EBLK
_NWORK = _NTILE + _E
_NW = 32
_CHUNK = 64
_NCH = _N // _NW // _CHUNK
_OW = _D + 128
_NEG = -1e30


def _layernorm(v, g, b, eps=1e-5):
    m = jnp.mean(v, axis=-1, keepdims=True)
    c = v - m
    var = jnp.mean(c * c, axis=-1, keepdims=True)
    return c * jax.lax.rsqrt(var + eps) * g + b


def _gelu_exact(v):
    return 0.5 * v * (1.0 + jax.lax.erf(v * float(1.0 / np.sqrt(2.0))))


def _act1(kind, h):
    if kind == 'arith':
        return jax.nn.relu(h)
    if kind == 'quantum':
        return jnp.tanh(h)
    return _gelu_exact(h)


def _act2(kind, o):
    if kind == 'quantum':
        return jnp.tanh(o)
    if kind == 'geom':
        return _gelu_exact(o)
    return o


def _dot_t(a, b):
    return jax.lax.dot_general(a, b, (((1,), (1,)), ((), ())),
                               preferred_element_type=jnp.float32)


def _dot(a, b):
    return jax.lax.dot_general(a, b, (((1,), (0,)), ((), ())),
                               preferred_element_type=jnp.float32)


def _gate_kernel(x_ref, wg_ref, bg_ref, e0_ref, e1_ref, g0_ref, g1_ref):
    x = x_ref[...]
    logits = _dot_t(x, wg_ref[...]) + bg_ref[...]
    col = jax.lax.broadcasted_iota(jnp.int32, logits.shape, 1)
    valid = col < _E
    lm = jnp.where(valid, logits, _NEG)
    m0 = jnp.max(lm, axis=1, keepdims=True)
    is0 = jnp.logical_and(lm == m0, valid)
    arg0 = jnp.min(jnp.where(is0, col, 127), axis=1, keepdims=True)
    lm1 = jnp.where(col == arg0, _NEG, lm)
    m1 = jnp.max(lm1, axis=1, keepdims=True)
    is1 = jnp.logical_and(lm1 == m1, jnp.logical_and(valid, col != arg0))
    arg1 = jnp.min(jnp.where(is1, col, 127), axis=1, keepdims=True)
    g0 = 1.0 / (1.0 + jnp.exp((m1 - m0) * _SQRT_PHI))
    e0_ref[...] = arg0
    e1_ref[...] = arg1
    g0_ref[...] = g0
    g1_ref[...] = 1.0 - g0


def _gate(x2d, wg, bg):
    grid = (_N // _TBLK,)
    spec_c1 = pl.BlockSpec((_TBLK, 1), lambda i: (i, 0))
    return pl.pallas_call(
        _gate_kernel,
        grid=grid,
        in_specs=[pl.BlockSpec((_TBLK, _D), lambda i: (i, 0)),
                  pl.BlockSpec(wg.shape, lambda i: (0, 0)),
                  pl.BlockSpec(bg.shape, lambda i: (0, 0))],
        out_specs=[spec_c1] * 4,
        out_shape=[jax.ShapeDtypeStruct((_N, 1), jnp.int32),
                   jax.ShapeDtypeStruct((_N, 1), jnp.int32),
                   jax.ShapeDtypeStruct((_N, 1), jnp.float32),
                   jax.ShapeDtypeStruct((_N, 1), jnp.float32)],
    )(x2d, wg, bg)


def _plan_kernel(e0_ref, e1_ref, pos0_ref, pos1_ref, work_ref):
    e0 = e0_ref[...]
    e1 = e1_ref[...]
    rr, cc = e0.shape
    ui = jax.lax.broadcasted_iota(jnp.int32, (cc, cc), 0)
    uj = jax.lax.broadcasted_iota(jnp.int32, (cc, cc), 1)
    upper = (ui < uj).astype(jnp.float32)
    vi = jax.lax.broadcasted_iota(jnp.int32, (rr, rr), 0)
    vj = jax.lax.broadcasted_iota(jnp.int32, (rr, rr), 1)
    lower = (vj < vi).astype(jnp.float32)

    base = jnp.zeros((1, 1), jnp.float32)
    pos0 = jnp.zeros((rr, cc), jnp.float32)
    pos1 = jnp.zeros((rr, cc), jnp.float32)
    starts, ends = [], []
    for e in range(_E):
        hit0 = (e0 == e)
        hit1 = (e1 == e)
        cnt = hit0.astype(jnp.float32) + hit1.astype(jnp.float32)
        excl = _dot(cnt, upper)
        rowsum = jnp.sum(cnt, axis=1, keepdims=True)
        carry = _dot(lower, rowsum)
        gpos = base + excl + carry
        pos0 = pos0 + hit0.astype(jnp.float32) * gpos
        pos1 = pos1 + hit1.astype(jnp.float32) * gpos
        total = jnp.sum(rowsum, axis=0, keepdims=True)
        starts.append(base)
        base = base + total
        ends.append(base)
    pos0_ref[...] = pos0.astype(jnp.int32)
    pos1_ref[...] = pos1.astype(jnp.int32)

    wcol = jax.lax.broadcasted_iota(jnp.int32, (1, 128), 1)
    ew = jnp.full((1, 128), _E + 1, jnp.int32)
    mw = jnp.full((1, 128), _NTILE - 1, jnp.int32)
    sw = jnp.zeros((1, 128), jnp.int32)
    qw = jnp.zeros((1, 128), jnp.int32)
    running = jnp.zeros((1, 1), jnp.int32)
    for e in range(_E):
        s_i = starts[e].astype(jnp.int32)
        q_i = ends[e].astype(jnp.int32)
        first_t = lax.div(s_i, _EBLK)
        last_t = lax.div(q_i - 1, _EBLK)
        count = jnp.where(q_i > s_i, last_t - first_t + 1, 0)
        sel = jnp.logical_and(wcol >= running, wcol < running + count)
        ew = jnp.where(sel, e, ew)
        mw = jnp.where(sel, first_t + (wcol - running), mw)
        sw = jnp.where(sel, s_i, sw)
        qw = jnp.where(sel, q_i, qw)
        running = running + count
    work_ref[0:1, :] = ew
    work_ref[1:2, :] = mw
    work_ref[2:3, :] = sw
    work_ref[3:4, :] = qw


def _plan(e0m, e1m):
    return pl.pallas_call(
        _plan_kernel,
        out_shape=[jax.ShapeDtypeStruct(e0m.shape, jnp.int32),
                   jax.ShapeDtypeStruct(e0m.shape, jnp.int32),
                   jax.ShapeDtypeStruct((4, 128), jnp.int32)],
    )(e0m, e1m)


def _dispatch_sc(x2d, p0w, p1w):
    mesh = plsc.VectorSubcoreMesh(core_axis_name="c", subcore_axis_name="s")

    @functools.partial(
        pl.kernel,
        out_type=jax.ShapeDtypeStruct((_NPAIR, _D), jnp.float32),
        mesh=mesh,
        scratch_types=[
            pltpu.VMEM((_NCH, _CHUNK), jnp.int32),
            pltpu.VMEM((_NCH, _CHUNK), jnp.int32),
            pltpu.VMEM((_CHUNK, _D), jnp.float32),
            pltpu.SemaphoreType.DMA,
        ],
    )
    def k(x_hbm, p0_hbm, p1_hbm, out_hbm, idx0_v, idx1_v, rows_v, sem):
        wid = lax.axis_index("s") * 2 + lax.axis_index("c")
        base = wid * (_N // _NW)
        pltpu.sync_copy(p0_hbm.at[wid], idx0_v)
        pltpu.sync_copy(p1_hbm.at[wid], idx1_v)
        for j in range(_NCH):
            pltpu.sync_copy(x_hbm.at[pl.ds(base + j * _CHUNK, _CHUNK)], rows_v)
            pltpu.async_copy(rows_v, out_hbm.at[idx0_v.at[j]], sem).wait()
            pltpu.async_copy(rows_v, out_hbm.at[idx1_v.at[j]], sem).wait()

    return k(x2d, p0w, p1w)


def _experts_kernel(eparams, ew, mw, sw, qw, xs_ref, out_ref):
    i = pl.program_id(0)
    e_id = ew[i]
    m = mw[i]
    prev = mw[jnp.maximum(i - 1, 0)]
    first = jnp.logical_or(i == 0, m != prev)

    @pl.when(first)
    def _():
        out_ref[...] = jnp.zeros_like(out_ref)

    rows = m * _EBLK + jax.lax.broadcasted_iota(jnp.int32, (_EBLK, 1), 0)
    mask = jnp.logical_and(rows >= sw[i], rows < qw[i]).astype(jnp.float32)
    x = xs_ref[...]
    for e in range(_E):
        @pl.when(e_id == e)
        def _(e=e):
            w1, b1, w2, b2, wc1, bc1, wc2, bc2, spec, lng, lnb = eparams[e]
            z = x + spec[...]
            h = _act1(_KINDS[e], _dot_t(z, w1[...]) + b1[...])
            o = _act2(_KINDS[e], _dot_t(h, w2[...]) + b2[...])
            o = _layernorm(o, lng[...], lnb[...])
            r = jax.nn.relu(_dot_t(o, wc1[...]) + bc1[...])
            clin = jnp.sum(r * wc2[...], axis=1, keepdims=True) + bc2[...]
            ce = jax.nn.sigmoid(clin)
            out_ref[:, :_D] += mask * o
            out_ref[:, _D:] += mask * jnp.broadcast_to(ce, (_EBLK, _OW - _D))


def _experts(xs, flat, ew, mw, sw, qw):
    nflat = len(flat)

    def body(ew_r, mw_r, sw_r, qw_r, xs_ref, *refs):
        eparams = [refs[11 * e:11 * e + 11] for e in range(_E)]
        _experts_kernel(eparams, ew_r, mw_r, sw_r, qw_r, xs_ref, refs[nflat])

    grid_spec = pltpu.PrefetchScalarGridSpec(
        num_scalar_prefetch=4,
        grid=(_NWORK,),
        in_specs=[pl.BlockSpec((_EBLK, _D), lambda i, ew, mw, sw, qw: (mw[i], 0))]
        + [pl.BlockSpec(w.shape, lambda i, *_: (0,) * w.ndim) for w in flat],
        out_specs=pl.BlockSpec((_EBLK, _OW), lambda i, ew, mw, sw, qw: (mw[i], 0)),
    )
    return pl.pallas_call(
        body,
        grid_spec=grid_spec,
        out_shape=jax.ShapeDtypeStruct((_NPAIR, _OW), jnp.float32),
    )(ew, mw, sw, qw, xs, *flat)


def _combine_gather_sc(outs, p0w, p1w):
    mesh = plsc.VectorSubcoreMesh(core_axis_name="c", subcore_axis_name="s")

    @functools.partial(
        pl.kernel,
        out_type=(jax.ShapeDtypeStruct((_N, _OW), jnp.float32),
                  jax.ShapeDtypeStruct((_N, _OW), jnp.float32)),
        mesh=mesh,
        scratch_types=[
            pltpu.VMEM((_NCH, _CHUNK), jnp.int32),
            pltpu.VMEM((_NCH, _CHUNK), jnp.int32),
            pltpu.VMEM((_CHUNK, _OW), jnp.float32),
            pltpu.SemaphoreType.DMA,
        ],
    )
    def k(outs_hbm, p0_hbm, p1_hbm, g0_hbm, g1_hbm, idx0_v, idx1_v, rows_v, sem):
        wid = lax.axis_index("s") * 2 + lax.axis_index("c")
        base = wid * (_N // _NW)
        pltpu.sync_copy(p0_hbm.at[wid], idx0_v)
        pltpu.sync_copy(p1_hbm.at[wid], idx1_v)
        for j in range(_NCH):
            dst = pl.ds(base + j * _CHUNK, _CHUNK)
            pltpu.async_copy(outs_hbm.at[idx0_v.at[j]], rows_v, sem).wait()
            pltpu.sync_copy(rows_v, g0_hbm.at[dst])
            pltpu.async_copy(outs_hbm.at[idx1_v.at[j]], rows_v, sem).wait()
            pltpu.sync_copy(rows_v, g1_hbm.at[dst])

    return k(outs, p0w, p1w)


def _combine_kernel(r0_ref, r1_ref, g0_ref, g1_ref, wcm_ref, bcm_ref,
                    cg_ref, cb_ref, out_ref, conf_ref):
    g0 = g0_ref[...]
    g1 = g1_ref[...]
    r0 = r0_ref[...]
    r1 = r1_ref[...]
    combined = g0 * r0[:, :_D] + g1 * r1[:, :_D]
    conf = g0 * r0[:, _D:_D + 1] + g1 * r1[:, _D:_D + 1]
    y = _dot_t(combined, wcm_ref[...]) + bcm_ref[...]
    y = _layernorm(y, cg_ref[...], cb_ref[...])
    out_ref[...] = y
    conf_ref[...] = jnp.broadcast_to(conf, conf_ref.shape)


def _combine(g0rows, g1rows, gw0, gw1, wcm, bcm, cg, cb):
    grid = (_N // _TBLK,)
    spec_row = pl.BlockSpec((_TBLK, _OW), lambda i: (i, 0))
    spec_c1 = pl.BlockSpec((_TBLK, 1), lambda i: (i, 0))
    return pl.pallas_call(
        _combine_kernel,
        grid=grid,
        in_specs=[spec_row, spec_row, spec_c1, spec_c1,
                  pl.BlockSpec(wcm.shape, lambda i: (0, 0)),
                  pl.BlockSpec(bcm.shape, lambda i: (0, 0)),
                  pl.BlockSpec(cg.shape, lambda i: (0, 0)),
                  pl.BlockSpec(cb.shape, lambda i: (0, 0))],
        out_specs=[pl.BlockSpec((_TBLK, _D), lambda i: (i, 0)),
                   pl.BlockSpec((_TBLK, 128), lambda i: (i, 0))],
        out_shape=[jax.ShapeDtypeStruct((_N, _D), jnp.float32),
                   jax.ShapeDtypeStruct((_N, 128), jnp.float32)],
    )(g0rows, g1rows, gw0, gw1, wcm, bcm, cg, cb)


def kernel(x, params):
    b, s, d = x.shape
    x2d = x.reshape(b * s, d)
    gate = params['gate']
    wg = jnp.zeros((128, d), jnp.float32).at[:_E].set(gate['W'])
    bg = jnp.zeros((1, 128), jnp.float32).at[0, :_E].set(gate['b'])

    e0, e1, gw0, gw1 = _gate(x2d, wg, bg)
    pos0, pos1, work = _plan(e0.reshape(_N // 128, 128),
                             e1.reshape(_N // 128, 128))
    ew, mw, sw, qw = (work[j, :_NWORK] for j in range(4))
    p0w = pos0.reshape(_NW, _NCH, _CHUNK)
    p1w = pos1.reshape(_NW, _NCH, _CHUNK)

    xs = _dispatch_sc(x2d, p0w, p1w)

    flat = []
    for e in range(_E):
        p = params['experts'][e]
        flat += [p['W1'], p['b1'][None, :], p['W2'], p['b2'][None, :],
                 p['Wc1'], p['bc1'][None, :], p['Wc2'], p['bc2'][None, :],
                 p['spec'][None, :], p['ln_g'][None, :], p['ln_b'][None, :]]
    outs = _experts(xs, tuple(flat), ew, mw, sw, qw)

    g0rows, g1rows = _combine_gather_sc(outs, p0w, p1w)

    cmb = params['combiner']
    out2d, conf = _combine(g0rows, g1rows, gw0, gw1, cmb['W'],
                           cmb['b'][None, :], cmb['ln_g'][None, :],
                           cmb['ln_b'][None, :])
    return out2d.reshape(b, s, d), conf[:, 0].reshape(b, s)


# speedup vs baseline: 1.9799x; 1.0635x over previous
import functools

import jax
import jax.numpy as jnp
import numpy as np
from jax import lax
from jax.experimental import pallas as pl
from jax.experimental.pallas import tpu as pltpu
from jax.experimental.pallas import tpu_sc as plsc

_PHI = (1.0 + 5.0 ** 0.5) / 2.0
_SQRT_PHI = float(np.sqrt(_PHI))
_D = 768
_E = 6
_KINDS = ('arith', 'general', 'geom', 'quantum', 'general', 'general')
_N = 8192
_NPAIR = 2 * _N
_TBLK = 512
_EBLK = 512
_NTILE = _NPAIR // _EBLK
_NWORK = _NTILE + _E
_NW = 32
_CHUNK = 64
_NCH = _N // _NW // _CHUNK
_OW = _D + 128
_NEG = -1e30


def _layernorm(v, g, b, eps=1e-5):
    m = jnp.mean(v, axis=-1, keepdims=True)
    c = v - m
    var = jnp.mean(c * c, axis=-1, keepdims=True)
    return c * jax.lax.rsqrt(var + eps) * g + b


def _gelu_exact(v):
    return 0.5 * v * (1.0 + jax.lax.erf(v * float(1.0 / np.sqrt(2.0))))


def _act1(kind, h):
    if kind == 'arith':
        return jax.nn.relu(h)
    if kind == 'quantum':
        return jnp.tanh(h)
    return _gelu_exact(h)


def _act2(kind, o):
    if kind == 'quantum':
        return jnp.tanh(o)
    if kind == 'geom':
        return _gelu_exact(o)
    return o


def _dot_t(a, b):
    return jax.lax.dot_general(a, b, (((1,), (1,)), ((), ())),
                               preferred_element_type=jnp.float32)


def _dot(a, b):
    return jax.lax.dot_general(a, b, (((1,), (0,)), ((), ())),
                               preferred_element_type=jnp.float32)


def _gate_kernel(x_ref, wg_ref, bg_ref, e0_ref, e1_ref, g0_ref, g1_ref):
    x = x_ref[...]
    logits = _dot_t(x, wg_ref[...]) + bg_ref[...]
    col = jax.lax.broadcasted_iota(jnp.int32, logits.shape, 1)
    valid = col < _E
    lm = jnp.where(valid, logits, _NEG)
    m0 = jnp.max(lm, axis=1, keepdims=True)
    is0 = jnp.logical_and(lm == m0, valid)
    arg0 = jnp.min(jnp.where(is0, col, 127), axis=1, keepdims=True)
    lm1 = jnp.where(col == arg0, _NEG, lm)
    m1 = jnp.max(lm1, axis=1, keepdims=True)
    is1 = jnp.logical_and(lm1 == m1, jnp.logical_and(valid, col != arg0))
    arg1 = jnp.min(jnp.where(is1, col, 127), axis=1, keepdims=True)
    g0 = 1.0 / (1.0 + jnp.exp((m1 - m0) * _SQRT_PHI))
    e0_ref[...] = arg0
    e1_ref[...] = arg1
    g0_ref[...] = g0
    g1_ref[...] = 1.0 - g0


def _gate(x2d, wg, bg):
    grid = (_N // _TBLK,)
    spec_c1 = pl.BlockSpec((_TBLK, 1), lambda i: (i, 0))
    return pl.pallas_call(
        _gate_kernel,
        grid=grid,
        in_specs=[pl.BlockSpec((_TBLK, _D), lambda i: (i, 0)),
                  pl.BlockSpec(wg.shape, lambda i: (0, 0)),
                  pl.BlockSpec(bg.shape, lambda i: (0, 0))],
        out_specs=[spec_c1] * 4,
        out_shape=[jax.ShapeDtypeStruct((_N, 1), jnp.int32),
                   jax.ShapeDtypeStruct((_N, 1), jnp.int32),
                   jax.ShapeDtypeStruct((_N, 1), jnp.float32),
                   jax.ShapeDtypeStruct((_N, 1), jnp.float32)],
    )(x2d, wg, bg)


def _plan_kernel(e0_ref, e1_ref, pos0_ref, pos1_ref, work_ref):
    e0 = e0_ref[...]
    e1 = e1_ref[...]
    rr, cc = e0.shape
    ui = jax.lax.broadcasted_iota(jnp.int32, (cc, cc), 0)
    uj = jax.lax.broadcasted_iota(jnp.int32, (cc, cc), 1)
    upper = (ui < uj).astype(jnp.float32)
    vi = jax.lax.broadcasted_iota(jnp.int32, (rr, rr), 0)
    vj = jax.lax.broadcasted_iota(jnp.int32, (rr, rr), 1)
    lower = (vj < vi).astype(jnp.float32)

    base = jnp.zeros((1, 1), jnp.float32)
    pos0 = jnp.zeros((rr, cc), jnp.float32)
    pos1 = jnp.zeros((rr, cc), jnp.float32)
    starts, ends = [], []
    for e in range(_E):
        hit0 = (e0 == e)
        hit1 = (e1 == e)
        cnt = hit0.astype(jnp.float32) + hit1.astype(jnp.float32)
        excl = _dot(cnt, upper)
        rowsum = jnp.sum(cnt, axis=1, keepdims=True)
        carry = _dot(lower, rowsum)
        gpos = base + excl + carry
        pos0 = pos0 + hit0.astype(jnp.float32) * gpos
        pos1 = pos1 + hit1.astype(jnp.float32) * gpos
        total = jnp.sum(rowsum, axis=0, keepdims=True)
        starts.append(base)
        base = base + total
        ends.append(base)
    pos0_ref[...] = pos0.astype(jnp.int32)
    pos1_ref[...] = pos1.astype(jnp.int32)

    wcol = jax.lax.broadcasted_iota(jnp.int32, (1, 128), 1)
    ew = jnp.full((1, 128), _E + 1, jnp.int32)
    mw = jnp.full((1, 128), _NTILE - 1, jnp.int32)
    sw = jnp.zeros((1, 128), jnp.int32)
    qw = jnp.zeros((1, 128), jnp.int32)
    running = jnp.zeros((1, 1), jnp.int32)
    for e in range(_E):
        s_i = starts[e].astype(jnp.int32)
        q_i = ends[e].astype(jnp.int32)
        first_t = lax.div(s_i, _EBLK)
        last_t = lax.div(q_i - 1, _EBLK)
        count = jnp.where(q_i > s_i, last_t - first_t + 1, 0)
        sel = jnp.logical_and(wcol >= running, wcol < running + count)
        ew = jnp.where(sel, e, ew)
        mw = jnp.where(sel, first_t + (wcol - running), mw)
        sw = jnp.where(sel, s_i, sw)
        qw = jnp.where(sel, q_i, qw)
        running = running + count
    work_ref[0:1, :] = ew
    work_ref[1:2, :] = mw
    work_ref[2:3, :] = sw
    work_ref[3:4, :] = qw


def _plan(e0m, e1m):
    return pl.pallas_call(
        _plan_kernel,
        out_shape=[jax.ShapeDtypeStruct(e0m.shape, jnp.int32),
                   jax.ShapeDtypeStruct(e0m.shape, jnp.int32),
                   jax.ShapeDtypeStruct((4, 128), jnp.int32)],
    )(e0m, e1m)


def _dispatch_sc(x2d, p0w, p1w):
    mesh = plsc.VectorSubcoreMesh(core_axis_name="c", subcore_axis_name="s")

    @functools.partial(
        pl.kernel,
        out_type=jax.ShapeDtypeStruct((_NPAIR, _D), jnp.float32),
        mesh=mesh,
        scratch_types=[
            pltpu.VMEM((_NCH, _CHUNK), jnp.int32),
            pltpu.VMEM((_NCH, _CHUNK), jnp.int32),
            pltpu.VMEM((_CHUNK, _D), jnp.float32),
            pltpu.SemaphoreType.DMA,
        ],
    )
    def k(x_hbm, p0_hbm, p1_hbm, out_hbm, idx0_v, idx1_v, rows_v, sem):
        wid = lax.axis_index("s") * 2 + lax.axis_index("c")
        base = wid * (_N // _NW)
        pltpu.sync_copy(p0_hbm.at[wid], idx0_v)
        pltpu.sync_copy(p1_hbm.at[wid], idx1_v)
        for j in range(_NCH):
            pltpu.sync_copy(x_hbm.at[pl.ds(base + j * _CHUNK, _CHUNK)], rows_v)
            pltpu.async_copy(rows_v, out_hbm.at[idx0_v.at[j]], sem).wait()
            pltpu.async_copy(rows_v, out_hbm.at[idx1_v.at[j]], sem).wait()

    return k(x2d, p0w, p1w)


def _experts_kernel(eparams, ew, mw, sw, qw, xs_ref, out_ref):
    i = pl.program_id(0)
    e_id = ew[i]
    m = mw[i]
    prev = mw[jnp.maximum(i - 1, 0)]
    first = jnp.logical_or(i == 0, m != prev)

    @pl.when(first)
    def _():
        out_ref[...] = jnp.zeros_like(out_ref)

    rows = m * _EBLK + jax.lax.broadcasted_iota(jnp.int32, (_EBLK, 1), 0)
    mask = jnp.logical_and(rows >= sw[i], rows < qw[i]).astype(jnp.float32)
    x = xs_ref[...]
    for e in range(_E):
        @pl.when(e_id == e)
        def _(e=e):
            w1, b1, w2, b2, wc1, bc1, wc2, bc2, spec, lng, lnb = eparams[e]
            z = x + spec[...]
            h = _act1(_KINDS[e], _dot_t(z, w1[...]) + b1[...])
            o = _act2(_KINDS[e], _dot_t(h, w2[...]) + b2[...])
            o = _layernorm(o, lng[...], lnb[...])
            r = jax.nn.relu(_dot_t(o, wc1[...]) + bc1[...])
            clin = jnp.sum(r * wc2[...], axis=1, keepdims=True) + bc2[...]
            ce = jax.nn.sigmoid(clin)
            out_ref[:, :_D] += mask * o
            out_ref[:, _D:] += mask * jnp.broadcast_to(ce, (_EBLK, _OW - _D))


def _experts(xs, flat, ew, mw, sw, qw):
    nflat = len(flat)

    def body(ew_r, mw_r, sw_r, qw_r, xs_ref, *refs):
        eparams = [refs[11 * e:11 * e + 11] for e in range(_E)]
        _experts_kernel(eparams, ew_r, mw_r, sw_r, qw_r, xs_ref, refs[nflat])

    grid_spec = pltpu.PrefetchScalarGridSpec(
        num_scalar_prefetch=4,
        grid=(_NWORK,),
        in_specs=[pl.BlockSpec((_EBLK, _D), lambda i, ew, mw, sw, qw: (mw[i], 0))]
        + [pl.BlockSpec(w.shape, lambda i, *_: (0,) * w.ndim) for w in flat],
        out_specs=pl.BlockSpec((_EBLK, _OW), lambda i, ew, mw, sw, qw: (mw[i], 0)),
    )
    return pl.pallas_call(
        body,
        grid_spec=grid_spec,
        out_shape=jax.ShapeDtypeStruct((_NPAIR, _OW), jnp.float32),
    )(ew, mw, sw, qw, xs, *flat)


def _combine_gather_sc(outs, p0w, p1w):
    mesh = plsc.VectorSubcoreMesh(core_axis_name="c", subcore_axis_name="s")

    @functools.partial(
        pl.kernel,
        out_type=(jax.ShapeDtypeStruct((_N, _OW), jnp.float32),
                  jax.ShapeDtypeStruct((_N, _OW), jnp.float32)),
        mesh=mesh,
        scratch_types=[
            pltpu.VMEM((_NCH, _CHUNK), jnp.int32),
            pltpu.VMEM((_NCH, _CHUNK), jnp.int32),
            pltpu.VMEM((_CHUNK, _OW), jnp.float32),
            pltpu.SemaphoreType.DMA,
        ],
    )
    def k(outs_hbm, p0_hbm, p1_hbm, g0_hbm, g1_hbm, idx0_v, idx1_v, rows_v, sem):
        wid = lax.axis_index("s") * 2 + lax.axis_index("c")
        base = wid * (_N // _NW)
        pltpu.sync_copy(p0_hbm.at[wid], idx0_v)
        pltpu.sync_copy(p1_hbm.at[wid], idx1_v)
        for j in range(_NCH):
            dst = pl.ds(base + j * _CHUNK, _CHUNK)
            pltpu.async_copy(outs_hbm.at[idx0_v.at[j]], rows_v, sem).wait()
            pltpu.sync_copy(rows_v, g0_hbm.at[dst])
            pltpu.async_copy(outs_hbm.at[idx1_v.at[j]], rows_v, sem).wait()
            pltpu.sync_copy(rows_v, g1_hbm.at[dst])

    return k(outs, p0w, p1w)


def _combine_kernel(r0_ref, r1_ref, g0_ref, g1_ref, wcm_ref, bcm_ref,
                    cg_ref, cb_ref, out_ref, conf_ref):
    g0 = g0_ref[...]
    g1 = g1_ref[...]
    r0 = r0_ref[...]
    r1 = r1_ref[...]
    combined = g0 * r0[:, :_D] + g1 * r1[:, :_D]
    conf = g0 * r0[:, _D:_D + 1] + g1 * r1[:, _D:_D + 1]
    y = _dot_t(combined, wcm_ref[...]) + bcm_ref[...]
    y = _layernorm(y, cg_ref[...], cb_ref[...])
    out_ref[...] = y
    conf_ref[...] = jnp.broadcast_to(conf, conf_ref.shape)


def _combine(g0rows, g1rows, gw0, gw1, wcm, bcm, cg, cb):
    grid = (_N // _TBLK,)
    spec_row = pl.BlockSpec((_TBLK, _OW), lambda i: (i, 0))
    spec_c1 = pl.BlockSpec((_TBLK, 1), lambda i: (i, 0))
    return pl.pallas_call(
        _combine_kernel,
        grid=grid,
        in_specs=[spec_row, spec_row, spec_c1, spec_c1,
                  pl.BlockSpec(wcm.shape, lambda i: (0, 0)),
                  pl.BlockSpec(bcm.shape, lambda i: (0, 0)),
                  pl.BlockSpec(cg.shape, lambda i: (0, 0)),
                  pl.BlockSpec(cb.shape, lambda i: (0, 0))],
        out_specs=[pl.BlockSpec((_TBLK, _D), lambda i: (i, 0)),
                   pl.BlockSpec((_TBLK, 128), lambda i: (i, 0))],
        out_shape=[jax.ShapeDtypeStruct((_N, _D), jnp.float32),
                   jax.ShapeDtypeStruct((_N, 128), jnp.float32)],
    )(g0rows, g1rows, gw0, gw1, wcm, bcm, cg, cb)


def kernel(x, params):
    b, s, d = x.shape
    x2d = x.reshape(b * s, d)
    gate = params['gate']
    wg = jnp.zeros((128, d), jnp.float32).at[:_E].set(gate['W'])
    bg = jnp.zeros((1, 128), jnp.float32).at[0, :_E].set(gate['b'])

    e0, e1, gw0, gw1 = _gate(x2d, wg, bg)
    pos0, pos1, work = _plan(e0.reshape(_N // 128, 128),
                             e1.reshape(_N // 128, 128))
    ew, mw, sw, qw = (work[j, :_NWORK] for j in range(4))
    p0w = pos0.reshape(_NW, _NCH, _CHUNK)
    p1w = pos1.reshape(_NW, _NCH, _CHUNK)

    xs = _dispatch_sc(x2d, p0w, p1w)

    flat = []
    for e in range(_E):
        p = params['experts'][e]
        flat += [p['W1'], p['b1'][None, :], p['W2'], p['b2'][None, :],
                 p['Wc1'], p['bc1'][None, :], p['Wc2'], p['bc2'][None, :],
                 p['spec'][None, :], p['ln_g'][None, :], p['ln_b'][None, :]]
    outs = _experts(xs, tuple(flat), ew, mw, sw, qw)

    g0rows, g1rows = _combine_gather_sc(outs, p0w, p1w)

    cmb = params['combiner']
    out2d, conf = _combine(g0rows, g1rows, gw0, gw1, cmb['W'],
                           cmb['b'][None, :], cmb['ln_g'][None, :],
                           cmb['ln_b'][None, :])
    return out2d.reshape(b, s, d), conf[:, 0].reshape(b, s)


# speedup vs baseline: 2.0261x; 1.0233x over previous
import functools

import jax
import jax.numpy as jnp
import numpy as np
from jax import lax
from jax.experimental import pallas as pl
from jax.experimental.pallas import tpu as pltpu
from jax.experimental.pallas import tpu_sc as plsc

_PHI = (1.0 + 5.0 ** 0.5) / 2.0
_SQRT_PHI = float(np.sqrt(_PHI))
_D = 768
_E = 6
_KINDS = ('arith', 'general', 'geom', 'quantum', 'general', 'general')
_N = 8192
_NPAIR = 2 * _N
_TBLK = 1024
_EBLK = 512
_NTILE = _NPAIR // _EBLK
_NWORK = _NTILE + _E
_NW = 32
_CHUNK = 64
_NCH = _N // _NW // _CHUNK
_OW = _D + 128
_NEG = -1e30


def _layernorm(v, g, b, eps=1e-5):
    m = jnp.mean(v, axis=-1, keepdims=True)
    c = v - m
    var = jnp.mean(c * c, axis=-1, keepdims=True)
    return c * jax.lax.rsqrt(var + eps) * g + b


def _gelu_exact(v):
    return 0.5 * v * (1.0 + jax.lax.erf(v * float(1.0 / np.sqrt(2.0))))


def _act1(kind, h):
    if kind == 'arith':
        return jax.nn.relu(h)
    if kind == 'quantum':
        return jnp.tanh(h)
    return _gelu_exact(h)


def _act2(kind, o):
    if kind == 'quantum':
        return jnp.tanh(o)
    if kind == 'geom':
        return _gelu_exact(o)
    return o


def _dot_t(a, b):
    return jax.lax.dot_general(a, b, (((1,), (1,)), ((), ())),
                               preferred_element_type=jnp.float32)


def _dot(a, b):
    return jax.lax.dot_general(a, b, (((1,), (0,)), ((), ())),
                               preferred_element_type=jnp.float32)


def _gate_kernel(x_ref, wg_ref, bg_ref, e0_ref, e1_ref, g0_ref, g1_ref):
    x = x_ref[...]
    logits = _dot_t(x, wg_ref[...]) + bg_ref[...]
    col = jax.lax.broadcasted_iota(jnp.int32, logits.shape, 1)
    valid = col < _E
    lm = jnp.where(valid, logits, _NEG)
    m0 = jnp.max(lm, axis=1, keepdims=True)
    is0 = jnp.logical_and(lm == m0, valid)
    arg0 = jnp.min(jnp.where(is0, col, 127), axis=1, keepdims=True)
    lm1 = jnp.where(col == arg0, _NEG, lm)
    m1 = jnp.max(lm1, axis=1, keepdims=True)
    is1 = jnp.logical_and(lm1 == m1, jnp.logical_and(valid, col != arg0))
    arg1 = jnp.min(jnp.where(is1, col, 127), axis=1, keepdims=True)
    g0 = 1.0 / (1.0 + jnp.exp((m1 - m0) * _SQRT_PHI))
    e0_ref[...] = arg0
    e1_ref[...] = arg1
    g0_ref[...] = g0
    g1_ref[...] = 1.0 - g0


def _gate(x2d, wg, bg):
    grid = (_N // _TBLK,)
    spec_c1 = pl.BlockSpec((_TBLK, 1), lambda i: (i, 0))
    return pl.pallas_call(
        _gate_kernel,
        grid=grid,
        in_specs=[pl.BlockSpec((_TBLK, _D), lambda i: (i, 0)),
                  pl.BlockSpec(wg.shape, lambda i: (0, 0)),
                  pl.BlockSpec(bg.shape, lambda i: (0, 0))],
        out_specs=[spec_c1] * 4,
        out_shape=[jax.ShapeDtypeStruct((_N, 1), jnp.int32),
                   jax.ShapeDtypeStruct((_N, 1), jnp.int32),
                   jax.ShapeDtypeStruct((_N, 1), jnp.float32),
                   jax.ShapeDtypeStruct((_N, 1), jnp.float32)],
    )(x2d, wg, bg)


def _plan_kernel(e0_ref, e1_ref, pos0_ref, pos1_ref, work_ref):
    e0 = e0_ref[...]
    e1 = e1_ref[...]
    rr, cc = e0.shape
    ui = jax.lax.broadcasted_iota(jnp.int32, (cc, cc), 0)
    uj = jax.lax.broadcasted_iota(jnp.int32, (cc, cc), 1)
    upper = (ui < uj).astype(jnp.float32)
    vi = jax.lax.broadcasted_iota(jnp.int32, (rr, rr), 0)
    vj = jax.lax.broadcasted_iota(jnp.int32, (rr, rr), 1)
    lower = (vj < vi).astype(jnp.float32)

    base = jnp.zeros((1, 1), jnp.float32)
    pos0 = jnp.zeros((rr, cc), jnp.float32)
    pos1 = jnp.zeros((rr, cc), jnp.float32)
    starts, ends = [], []
    for e in range(_E):
        hit0 = (e0 == e)
        hit1 = (e1 == e)
        cnt = hit0.astype(jnp.float32) + hit1.astype(jnp.float32)
        excl = _dot(cnt, upper)
        rowsum = jnp.sum(cnt, axis=1, keepdims=True)
        carry = _dot(lower, rowsum)
        gpos = base + excl + carry
        pos0 = pos0 + hit0.astype(jnp.float32) * gpos
        pos1 = pos1 + hit1.astype(jnp.float32) * gpos
        total = jnp.sum(rowsum, axis=0, keepdims=True)
        starts.append(base)
        base = base + total
        ends.append(base)
    pos0_ref[...] = pos0.astype(jnp.int32)
    pos1_ref[...] = pos1.astype(jnp.int32)

    wcol = jax.lax.broadcasted_iota(jnp.int32, (1, 128), 1)
    ew = jnp.full((1, 128), _E + 1, jnp.int32)
    mw = jnp.full((1, 128), _NTILE - 1, jnp.int32)
    sw = jnp.zeros((1, 128), jnp.int32)
    qw = jnp.zeros((1, 128), jnp.int32)
    running = jnp.zeros((1, 1), jnp.int32)
    for e in range(_E):
        s_i = starts[e].astype(jnp.int32)
        q_i = ends[e].astype(jnp.int32)
        first_t = lax.div(s_i, _EBLK)
        last_t = lax.div(q_i - 1, _EBLK)
        count = jnp.where(q_i > s_i, last_t - first_t + 1, 0)
        sel = jnp.logical_and(wcol >= running, wcol < running + count)
        ew = jnp.where(sel, e, ew)
        mw = jnp.where(sel, first_t + (wcol - running), mw)
        sw = jnp.where(sel, s_i, sw)
        qw = jnp.where(sel, q_i, qw)
        running = running + count
    work_ref[0:1, :] = ew
    work_ref[1:2, :] = mw
    work_ref[2:3, :] = sw
    work_ref[3:4, :] = qw


def _plan(e0m, e1m):
    return pl.pallas_call(
        _plan_kernel,
        out_shape=[jax.ShapeDtypeStruct(e0m.shape, jnp.int32),
                   jax.ShapeDtypeStruct(e0m.shape, jnp.int32),
                   jax.ShapeDtypeStruct((4, 128), jnp.int32)],
    )(e0m, e1m)


def _dispatch_sc(x2d, p0w, p1w):
    mesh = plsc.VectorSubcoreMesh(core_axis_name="c", subcore_axis_name="s")

    @functools.partial(
        pl.kernel,
        out_type=jax.ShapeDtypeStruct((_NPAIR, _D), jnp.float32),
        mesh=mesh,
        scratch_types=[
            pltpu.VMEM((_NCH, _CHUNK), jnp.int32),
            pltpu.VMEM((_NCH, _CHUNK), jnp.int32),
            pltpu.VMEM((_CHUNK, _D), jnp.float32),
            pltpu.SemaphoreType.DMA,
        ],
    )
    def k(x_hbm, p0_hbm, p1_hbm, out_hbm, idx0_v, idx1_v, rows_v, sem):
        wid = lax.axis_index("s") * 2 + lax.axis_index("c")
        base = wid * (_N // _NW)
        pltpu.sync_copy(p0_hbm.at[wid], idx0_v)
        pltpu.sync_copy(p1_hbm.at[wid], idx1_v)
        for j in range(_NCH):
            pltpu.sync_copy(x_hbm.at[pl.ds(base + j * _CHUNK, _CHUNK)], rows_v)
            pltpu.async_copy(rows_v, out_hbm.at[idx0_v.at[j]], sem).wait()
            pltpu.async_copy(rows_v, out_hbm.at[idx1_v.at[j]], sem).wait()

    return k(x2d, p0w, p1w)


def _experts_kernel(eparams, ew, mw, sw, qw, xs_ref, out_ref):
    i = pl.program_id(0)
    e_id = ew[i]
    m = mw[i]
    prev = mw[jnp.maximum(i - 1, 0)]
    first = jnp.logical_or(i == 0, m != prev)

    @pl.when(first)
    def _():
        out_ref[...] = jnp.zeros_like(out_ref)

    rows = m * _EBLK + jax.lax.broadcasted_iota(jnp.int32, (_EBLK, 1), 0)
    mask = jnp.logical_and(rows >= sw[i], rows < qw[i]).astype(jnp.float32)
    x = xs_ref[...]
    for e in range(_E):
        @pl.when(e_id == e)
        def _(e=e):
            w1, b1, w2, b2, wc1, bc1, wc2, bc2, spec, lng, lnb = eparams[e]
            z = x + spec[...]
            h = _act1(_KINDS[e], _dot_t(z, w1[...]) + b1[...])
            o = _act2(_KINDS[e], _dot_t(h, w2[...]) + b2[...])
            o = _layernorm(o, lng[...], lnb[...])
            r = jax.nn.relu(_dot_t(o, wc1[...]) + bc1[...])
            clin = jnp.sum(r * wc2[...], axis=1, keepdims=True) + bc2[...]
            ce = jax.nn.sigmoid(clin)
            out_ref[:, :_D] += mask * o
            out_ref[:, _D:] += mask * jnp.broadcast_to(ce, (_EBLK, _OW - _D))


def _experts(xs, flat, ew, mw, sw, qw):
    nflat = len(flat)

    def body(ew_r, mw_r, sw_r, qw_r, xs_ref, *refs):
        eparams = [refs[11 * e:11 * e + 11] for e in range(_E)]
        _experts_kernel(eparams, ew_r, mw_r, sw_r, qw_r, xs_ref, refs[nflat])

    grid_spec = pltpu.PrefetchScalarGridSpec(
        num_scalar_prefetch=4,
        grid=(_NWORK,),
        in_specs=[pl.BlockSpec((_EBLK, _D), lambda i, ew, mw, sw, qw: (mw[i], 0))]
        + [pl.BlockSpec(w.shape, lambda i, *_: (0,) * w.ndim) for w in flat],
        out_specs=pl.BlockSpec((_EBLK, _OW), lambda i, ew, mw, sw, qw: (mw[i], 0)),
    )
    return pl.pallas_call(
        body,
        grid_spec=grid_spec,
        out_shape=jax.ShapeDtypeStruct((_NPAIR, _OW), jnp.float32),
    )(ew, mw, sw, qw, xs, *flat)


def _combine_gather_sc(outs, p0w, p1w):
    mesh = plsc.VectorSubcoreMesh(core_axis_name="c", subcore_axis_name="s")

    @functools.partial(
        pl.kernel,
        out_type=(jax.ShapeDtypeStruct((_N, _OW), jnp.float32),
                  jax.ShapeDtypeStruct((_N, _OW), jnp.float32)),
        mesh=mesh,
        scratch_types=[
            pltpu.VMEM((_NCH, _CHUNK), jnp.int32),
            pltpu.VMEM((_NCH, _CHUNK), jnp.int32),
            pltpu.VMEM((_CHUNK, _OW), jnp.float32),
            pltpu.SemaphoreType.DMA,
        ],
    )
    def k(outs_hbm, p0_hbm, p1_hbm, g0_hbm, g1_hbm, idx0_v, idx1_v, rows_v, sem):
        wid = lax.axis_index("s") * 2 + lax.axis_index("c")
        base = wid * (_N // _NW)
        pltpu.sync_copy(p0_hbm.at[wid], idx0_v)
        pltpu.sync_copy(p1_hbm.at[wid], idx1_v)
        for j in range(_NCH):
            dst = pl.ds(base + j * _CHUNK, _CHUNK)
            pltpu.async_copy(outs_hbm.at[idx0_v.at[j]], rows_v, sem).wait()
            pltpu.sync_copy(rows_v, g0_hbm.at[dst])
            pltpu.async_copy(outs_hbm.at[idx1_v.at[j]], rows_v, sem).wait()
            pltpu.sync_copy(rows_v, g1_hbm.at[dst])

    return k(outs, p0w, p1w)


def _combine_kernel(r0_ref, r1_ref, g0_ref, g1_ref, wcm_ref, bcm_ref,
                    cg_ref, cb_ref, out_ref, conf_ref):
    g0 = g0_ref[...]
    g1 = g1_ref[...]
    r0 = r0_ref[...]
    r1 = r1_ref[...]
    combined = g0 * r0[:, :_D] + g1 * r1[:, :_D]
    conf = g0 * r0[:, _D:_D + 1] + g1 * r1[:, _D:_D + 1]
    y = _dot_t(combined, wcm_ref[...]) + bcm_ref[...]
    y = _layernorm(y, cg_ref[...], cb_ref[...])
    out_ref[...] = y
    conf_ref[...] = jnp.broadcast_to(conf, conf_ref.shape)


def _combine(g0rows, g1rows, gw0, gw1, wcm, bcm, cg, cb):
    grid = (_N // _TBLK,)
    spec_row = pl.BlockSpec((_TBLK, _OW), lambda i: (i, 0))
    spec_c1 = pl.BlockSpec((_TBLK, 1), lambda i: (i, 0))
    return pl.pallas_call(
        _combine_kernel,
        grid=grid,
        in_specs=[spec_row, spec_row, spec_c1, spec_c1,
                  pl.BlockSpec(wcm.shape, lambda i: (0, 0)),
                  pl.BlockSpec(bcm.shape, lambda i: (0, 0)),
                  pl.BlockSpec(cg.shape, lambda i: (0, 0)),
                  pl.BlockSpec(cb.shape, lambda i: (0, 0))],
        out_specs=[pl.BlockSpec((_TBLK, _D), lambda i: (i, 0)),
                   pl.BlockSpec((_TBLK, 128), lambda i: (i, 0))],
        out_shape=[jax.ShapeDtypeStruct((_N, _D), jnp.float32),
                   jax.ShapeDtypeStruct((_N, 128), jnp.float32)],
    )(g0rows, g1rows, gw0, gw1, wcm, bcm, cg, cb)


def kernel(x, params):
    b, s, d = x.shape
    x2d = x.reshape(b * s, d)
    gate = params['gate']
    wg = jnp.zeros((128, d), jnp.float32).at[:_E].set(gate['W'])
    bg = jnp.zeros((1, 128), jnp.float32).at[0, :_E].set(gate['b'])

    e0, e1, gw0, gw1 = _gate(x2d, wg, bg)
    pos0, pos1, work = _plan(e0.reshape(_N // 128, 128),
                             e1.reshape(_N // 128, 128))
    ew, mw, sw, qw = (work[j, :_NWORK] for j in range(4))
    p0w = pos0.reshape(_NW, _NCH, _CHUNK)
    p1w = pos1.reshape(_NW, _NCH, _CHUNK)

    xs = _dispatch_sc(x2d, p0w, p1w)

    flat = []
    for e in range(_E):
        p = params['experts'][e]
        flat += [p['W1'], p['b1'][None, :], p['W2'], p['b2'][None, :],
                 p['Wc1'], p['bc1'][None, :], p['Wc2'], p['bc2'][None, :],
                 p['spec'][None, :], p['ln_g'][None, :], p['ln_b'][None, :]]
    outs = _experts(xs, tuple(flat), ew, mw, sw, qw)

    g0rows, g1rows = _combine_gather_sc(outs, p0w, p1w)

    cmb = params['combiner']
    out2d, conf = _combine(g0rows, g1rows, gw0, gw1, cmb['W'],
                           cmb['b'][None, :], cmb['ln_g'][None, :],
                           cmb['ln_b'][None, :])
    return out2d.reshape(b, s, d), conf[:, 0].reshape(b, s)
